# baseline (speedup 1.0000x reference)
import functools

import jax
import jax.numpy as jnp
from jax import lax
from jax.experimental import pallas as pl
from jax.experimental.pallas import tpu as pltpu
from jax.experimental.pallas import tpu_sc as plsc

_S = 2048
_H = 1024
_I = 4096
_E = 8
_K = 2

_BLK = 2048
_NROWS = _E * _BLK
_SUB = 256
_NSUB = _BLK // _SUB
_IBLK = 512
_NI = _I // _IBLK

_NC, _NS = 2, 16
_NW = _NC * _NS
_TPW = _S // _NW
_CH = 32


def _router_body(xf_ref, rw_ref, rb_ref, pos1_ref, pos2_ref, w1b_ref,
                 w2b_ref, cnt_ref):
    xf = xf_ref[...]
    rw = rw_ref[...]
    rb = rb_ref[...]
    logits = lax.dot_general(xf, rw, (((1,), (1,)), ((), ())),
                             preferred_element_type=jnp.float32) + rb
    m = jnp.max(logits, axis=1, keepdims=True)
    ex = jnp.exp(logits - m)
    sm = ex / jnp.sum(ex, axis=1, keepdims=True)

    lane_e = lax.broadcasted_iota(jnp.int32, (_S, _E), 1)
    v1 = jnp.max(sm, axis=1, keepdims=True)
    i1 = jnp.min(jnp.where(sm == v1, lane_e, _E), axis=1, keepdims=True)
    sm2 = jnp.where(lane_e == i1, -1.0, sm)
    v2 = jnp.max(sm2, axis=1, keepdims=True)
    i2 = jnp.min(jnp.where(sm2 == v2, lane_e, _E), axis=1, keepdims=True)
    den = v1 + v2
    w1b_ref[...] = jnp.broadcast_to(v1 / den, (_S, 16))
    w2b_ref[...] = jnp.broadcast_to(v2 / den, (_S, 16))

    oh1 = (lane_e == i1)
    oh2 = (lane_e == i2)
    oh = (oh1 | oh2).astype(jnp.float32)

    rows = lax.broadcasted_iota(jnp.int32, (128, 128), 0)
    cols = lax.broadcasted_iota(jnp.int32, (128, 128), 1)
    tstrict = (rows > cols).astype(jnp.float32)
    parts = []
    carry = jnp.zeros((1, _E), jnp.float32)
    for b in range(_S // 128):
        blk = oh[b * 128:(b + 1) * 128, :]
        exc = lax.dot_general(tstrict, blk, (((1,), (0,)), ((), ())),
                              preferred_element_type=jnp.float32)
        parts.append(exc + carry)
        carry = carry + jnp.sum(blk, axis=0, keepdims=True)
    cum = jnp.concatenate(parts, axis=0)
    cnt_ref[...] = carry.astype(jnp.int32)

    off_row = (lane_e[0:1, :] * _BLK).astype(jnp.float32)
    dest = off_row + cum
    pos1 = jnp.sum(jnp.where(oh1, dest, 0.0), axis=1, keepdims=True)
    pos2 = jnp.sum(jnp.where(oh2, dest, 0.0), axis=1, keepdims=True)
    pos1_ref[...] = pos1.astype(jnp.int32)
    pos2_ref[...] = pos2.astype(jnp.int32)


def _router(xf, router_w, router_b):
    return pl.pallas_call(
        _router_body,
        out_shape=[
            jax.ShapeDtypeStruct((_S, 1), jnp.int32),
            jax.ShapeDtypeStruct((_S, 1), jnp.int32),
            jax.ShapeDtypeStruct((_S, 16), jnp.float32),
            jax.ShapeDtypeStruct((_S, 16), jnp.float32),
            jax.ShapeDtypeStruct((1, _E), jnp.int32),
        ],
    )(xf, router_w, router_b.reshape(1, _E))


def _dispatch_body(xf_hbm, pos1_hbm, pos2_hbm, xs_hbm, rows_v, idx1_v,
                   idx2_v, sem):
    wid = lax.axis_index("s") * _NC + lax.axis_index("c")
    base = wid * _TPW
    pltpu.sync_copy(xf_hbm.at[pl.ds(base, _TPW)], rows_v)
    pltpu.sync_copy(pos1_hbm.at[pl.ds(base, _TPW)], idx1_v)
    pltpu.sync_copy(pos2_hbm.at[pl.ds(base, _TPW)], idx2_v)
    pltpu.async_copy(rows_v, xs_hbm.at[idx1_v], sem).wait()
    pltpu.async_copy(rows_v, xs_hbm.at[idx2_v], sem).wait()


def _dispatch(xf, pos1, pos2):
    mesh = plsc.VectorSubcoreMesh(core_axis_name="c", subcore_axis_name="s",
                                  num_cores=_NC, num_subcores=_NS)
    return pl.kernel(
        _dispatch_body,
        out_type=jax.ShapeDtypeStruct((_NROWS, _H), jnp.float32),
        mesh=mesh,
        scratch_types=[
            pltpu.VMEM((_TPW, _H), jnp.float32),
            pltpu.VMEM((_TPW,), jnp.int32),
            pltpu.VMEM((_TPW,), jnp.int32),
            pltpu.SemaphoreType.DMA,
        ],
    )(xf, pos1, pos2)


def _ffn_body(cnt_ref, xs_ref, gw_ref, uw_ref, dw_ref, out_ref):
    e = pl.program_id(0)
    j = pl.program_id(1)
    c = cnt_ref[e]
    gw = gw_ref[0].astype(jnp.bfloat16)
    uw = uw_ref[0].astype(jnp.bfloat16)
    dw = dw_ref[0].astype(jnp.bfloat16)
    for s in range(_NSUB):
        @pl.when(c > s * _SUB)
        def _():
            sl = pl.ds(s * _SUB, _SUB)
            xb = xs_ref[sl, :].astype(jnp.bfloat16)
            g = lax.dot_general(xb, gw, (((1,), (1,)), ((), ())),
                                preferred_element_type=jnp.float32)
            u = lax.dot_general(xb, uw, (((1,), (1,)), ((), ())),
                                preferred_element_type=jnp.float32)
            h = (g * lax.logistic(g) * u).astype(jnp.bfloat16)
            p = lax.dot_general(h, dw, (((1,), (1,)), ((), ())),
                                preferred_element_type=jnp.float32)

            @pl.when(j == 0)
            def _():
                out_ref[sl, :] = p

            @pl.when(j > 0)
            def _():
                out_ref[sl, :] = out_ref[sl, :] + p


def _ffn(cnt, xs, gate_w, up_w, down_w):
    grid_spec = pltpu.PrefetchScalarGridSpec(
        num_scalar_prefetch=1,
        grid=(_E, _NI),
        in_specs=[
            pl.BlockSpec((_BLK, _H), lambda e, j, cnt: (e, 0)),
            pl.BlockSpec((1, _IBLK, _H), lambda e, j, cnt: (e, j, 0)),
            pl.BlockSpec((1, _IBLK, _H), lambda e, j, cnt: (e, j, 0)),
            pl.BlockSpec((1, _H, _IBLK), lambda e, j, cnt: (e, 0, j)),
        ],
        out_specs=pl.BlockSpec((_BLK, _H), lambda e, j, cnt: (e, 0)),
    )
    return pl.pallas_call(
        _ffn_body,
        grid_spec=grid_spec,
        out_shape=jax.ShapeDtypeStruct((_NROWS, _H), jnp.float32),
        compiler_params=pltpu.CompilerParams(
            dimension_semantics=("arbitrary", "arbitrary")),
    )(cnt, xs, gate_w, up_w, down_w)


def _combine_body(outs_hbm, pos1_hbm, pos2_hbm, w1b_hbm, w2b_hbm, fin_hbm,
                  rows1_v, rows2_v, acc_v, idx1_v, idx2_v, w1_v, w2_v, sem):
    wid = lax.axis_index("s") * _NC + lax.axis_index("c")
    for c in range(_TPW // _CH):
        base = wid * _TPW + c * _CH
        pltpu.sync_copy(pos1_hbm.at[pl.ds(base, _CH)], idx1_v)
        pltpu.sync_copy(pos2_hbm.at[pl.ds(base, _CH)], idx2_v)
        pltpu.sync_copy(w1b_hbm.at[pl.ds(base, _CH)], w1_v)
        pltpu.sync_copy(w2b_hbm.at[pl.ds(base, _CH)], w2_v)
        cp1 = pltpu.async_copy(outs_hbm.at[idx1_v], rows1_v, sem)
        cp1.wait()
        cp2 = pltpu.async_copy(outs_hbm.at[idx2_v], rows2_v, sem)
        cp2.wait()

        def body(i, carry):
            wv1 = w1_v[i, :]
            wv2 = w2_v[i, :]
            for l in range(_H // 16):
                sl = pl.ds(l * 16, 16)
                acc_v[i, sl] = wv1 * rows1_v[i, sl] + wv2 * rows2_v[i, sl]
            return carry

        lax.fori_loop(0, _CH, body, 0)
        pltpu.sync_copy(acc_v, fin_hbm.at[pl.ds(base, _CH)])


def _combine(outs, pos1, pos2, w1b, w2b):
    mesh = plsc.VectorSubcoreMesh(core_axis_name="c", subcore_axis_name="s",
                                  num_cores=_NC, num_subcores=_NS)
    return pl.kernel(
        _combine_body,
        out_type=jax.ShapeDtypeStruct((_S, _H), jnp.float32),
        mesh=mesh,
        scratch_types=[
            pltpu.VMEM((_CH, _H), jnp.float32),
            pltpu.VMEM((_CH, _H), jnp.float32),
            pltpu.VMEM((_CH, _H), jnp.float32),
            pltpu.VMEM((_CH,), jnp.int32),
            pltpu.VMEM((_CH,), jnp.int32),
            pltpu.VMEM((_CH, 16), jnp.float32),
            pltpu.VMEM((_CH, 16), jnp.float32),
            pltpu.SemaphoreType.DMA,
        ],
    )(outs, pos1, pos2, w1b, w2b)


@jax.jit
def kernel(x, router_w, router_b, gate_w, up_w, down_w):
    b, s, h = x.shape
    xf = x.reshape(s, h)
    pos1, pos2, w1b, w2b, cnt = _router(xf, router_w, router_b)
    pos1 = pos1.reshape(_S)
    pos2 = pos2.reshape(_S)
    cnt = cnt.reshape(_E)
    xs = _dispatch(xf, pos1, pos2)
    outs = _ffn(cnt, xs, gate_w, up_w, down_w)
    fin = _combine(outs, pos1, pos2, w1b, w2b)
    return fin.reshape(b, s, h)


# speedup vs baseline: 1.1407x; 1.1407x over previous
import functools

import jax
import jax.numpy as jnp
from jax import lax
from jax.experimental import pallas as pl
from jax.experimental.pallas import tpu as pltpu
from jax.experimental.pallas import tpu_sc as plsc

_S = 2048
_H = 1024
_I = 4096
_E = 8
_K = 2

_BLK = 1024
_BLK_SHIFT = 10
_NB = 12
_NROWS = _NB * _BLK
_SUB = 512
_IBLK = 1024
_NI = _I // _IBLK

_NC, _NS = 2, 16
_NW = _NC * _NS
_TPW = _S // _NW
_CH = 32


def _router_body(xf_ref, rw_ref, rb_ref, pos1_ref, pos2_ref, w1b_ref,
                 w2b_ref, be_ref, vr_ref):
    xf = xf_ref[...]
    rw = rw_ref[...]
    rb = rb_ref[...]
    logits = lax.dot_general(xf, rw, (((1,), (1,)), ((), ())),
                             preferred_element_type=jnp.float32) + rb
    m = jnp.max(logits, axis=1, keepdims=True)
    ex = jnp.exp(logits - m)
    sm = ex / jnp.sum(ex, axis=1, keepdims=True)

    lane_e = lax.broadcasted_iota(jnp.int32, (_S, _E), 1)
    v1 = jnp.max(sm, axis=1, keepdims=True)
    i1 = jnp.min(jnp.where(sm == v1, lane_e, _E), axis=1, keepdims=True)
    sm2 = jnp.where(lane_e == i1, -1.0, sm)
    v2 = jnp.max(sm2, axis=1, keepdims=True)
    i2 = jnp.min(jnp.where(sm2 == v2, lane_e, _E), axis=1, keepdims=True)
    den = v1 + v2
    w1b_ref[...] = jnp.broadcast_to(v1 / den, (_S, 16))
    w2b_ref[...] = jnp.broadcast_to(v2 / den, (_S, 16))

    oh1 = (lane_e == i1)
    oh2 = (lane_e == i2)
    oh = (oh1 | oh2).astype(jnp.float32)

    rows = lax.broadcasted_iota(jnp.int32, (128, 128), 0)
    cols = lax.broadcasted_iota(jnp.int32, (128, 128), 1)
    tstrict = (rows > cols).astype(jnp.float32)
    parts = []
    carry = jnp.zeros((1, _E), jnp.float32)
    for b in range(_S // 128):
        blk = oh[b * 128:(b + 1) * 128, :]
        exc = lax.dot_general(tstrict, blk, (((1,), (0,)), ((), ())),
                              preferred_element_type=jnp.float32)
        parts.append(exc + carry)
        carry = carry + jnp.sum(blk, axis=0, keepdims=True)
    cum = jnp.concatenate(parts, axis=0)
    counts_i = carry.astype(jnp.int32)

    nb = (counts_i + (_BLK - 1)) >> _BLK_SHIFT
    nb_f = nb.astype(jnp.float32)
    e_r = lax.broadcasted_iota(jnp.int32, (_E, _E), 0)
    e_c = lax.broadcasted_iota(jnp.int32, (_E, _E), 1)
    u_lt = (e_r < e_c).astype(jnp.float32)
    u_le = (e_r <= e_c).astype(jnp.float32)
    cum_excl = lax.dot_general(nb_f, u_lt, (((1,), (0,)), ((), ())),
                               preferred_element_type=jnp.float32)
    cum_incl = lax.dot_general(nb_f, u_le, (((1,), (0,)), ((), ())),
                               preferred_element_type=jnp.float32)
    off_row = cum_excl * float(_BLK)

    dest = off_row + cum
    pos1 = jnp.sum(jnp.where(oh1, dest, 0.0), axis=1, keepdims=True)
    pos2 = jnp.sum(jnp.where(oh2, dest, 0.0), axis=1, keepdims=True)
    pos1_ref[...] = pos1.astype(jnp.int32)
    pos2_ref[...] = pos2.astype(jnp.int32)

    b_iota = lax.broadcasted_iota(jnp.int32, (1, 128), 1)
    cum_incl_i = cum_incl.astype(jnp.int32)
    be_acc = jnp.zeros((1, 128), jnp.int32)
    for e in range(_E):
        be_acc = be_acc + jnp.where(b_iota >= cum_incl_i[:, e:e + 1], 1, 0)
    lane8 = lax.broadcasted_iota(jnp.int32, (1, _E), 1)
    last_e = jnp.max(jnp.where(counts_i > 0, lane8, 0), axis=1, keepdims=True)
    used = cum_incl_i[:, _E - 1:_E]
    be = jnp.minimum(be_acc, last_e)
    be_ref[...] = jnp.where(b_iota == _NB, used, be)

    seg_end = off_row + carry
    seg_end_b = jnp.zeros((1, 128), jnp.float32)
    for e in range(_E):
        seg_end_b = seg_end_b + jnp.where(be == e, seg_end[:, e:e + 1], 0.0)
    vr = jnp.clip(seg_end_b - (b_iota * _BLK).astype(jnp.float32),
                  0.0, float(_BLK))
    vr_ref[...] = vr.astype(jnp.int32)


def _router(xf, router_w, router_b):
    return pl.pallas_call(
        _router_body,
        out_shape=[
            jax.ShapeDtypeStruct((_S, 1), jnp.int32),
            jax.ShapeDtypeStruct((_S, 1), jnp.int32),
            jax.ShapeDtypeStruct((_S, 16), jnp.float32),
            jax.ShapeDtypeStruct((_S, 16), jnp.float32),
            jax.ShapeDtypeStruct((1, 128), jnp.int32),
            jax.ShapeDtypeStruct((1, 128), jnp.int32),
        ],
    )(xf, router_w, router_b.reshape(1, _E))


def _dispatch_body(xf_hbm, pos1_hbm, pos2_hbm, xs_hbm, rows_v, idx1_v,
                   idx2_v, sem):
    wid = lax.axis_index("s") * _NC + lax.axis_index("c")
    base = wid * _TPW
    pltpu.sync_copy(xf_hbm.at[pl.ds(base, _TPW)], rows_v)
    pltpu.sync_copy(pos1_hbm.at[pl.ds(base, _TPW)], idx1_v)
    pltpu.sync_copy(pos2_hbm.at[pl.ds(base, _TPW)], idx2_v)
    pltpu.async_copy(rows_v, xs_hbm.at[idx1_v], sem).wait()
    pltpu.async_copy(rows_v, xs_hbm.at[idx2_v], sem).wait()


def _dispatch(xf, pos1, pos2):
    mesh = plsc.VectorSubcoreMesh(core_axis_name="c", subcore_axis_name="s",
                                  num_cores=_NC, num_subcores=_NS)
    return pl.kernel(
        _dispatch_body,
        out_type=jax.ShapeDtypeStruct((_NROWS, _H), jnp.float32),
        mesh=mesh,
        scratch_types=[
            pltpu.VMEM((_TPW, _H), jnp.float32),
            pltpu.VMEM((_TPW,), jnp.int32),
            pltpu.VMEM((_TPW,), jnp.int32),
            pltpu.SemaphoreType.DMA,
        ],
    )(xf, pos1, pos2)


def _ffn_body(be_ref, vr_ref, xs_ref, gw_ref, uw_ref, dw_ref, out_ref):
    b = pl.program_id(0)
    j = pl.program_id(1)
    vr = vr_ref[b]
    gw = gw_ref[0].astype(jnp.bfloat16)
    uw = uw_ref[0].astype(jnp.bfloat16)
    dw = dw_ref[0].astype(jnp.bfloat16)
    for s in range(_BLK // _SUB):
        @pl.when(vr > s * _SUB)
        def _():
            sl = pl.ds(s * _SUB, _SUB)
            xb = xs_ref[sl, :].astype(jnp.bfloat16)
            g = lax.dot_general(xb, gw, (((1,), (1,)), ((), ())),
                                preferred_element_type=jnp.float32)
            u = lax.dot_general(xb, uw, (((1,), (1,)), ((), ())),
                                preferred_element_type=jnp.float32)
            h = (g * lax.logistic(g) * u).astype(jnp.bfloat16)
            p = lax.dot_general(h, dw, (((1,), (1,)), ((), ())),
                                preferred_element_type=jnp.float32)

            @pl.when(j == 0)
            def _():
                out_ref[sl, :] = p

            @pl.when(j > 0)
            def _():
                out_ref[sl, :] = out_ref[sl, :] + p


def _ffn(be, vr, xs, gate_w, up_w, down_w):
    grid_spec = pltpu.PrefetchScalarGridSpec(
        num_scalar_prefetch=2,
        grid=(_NB, _NI),
        in_specs=[
            pl.BlockSpec((_BLK, _H), lambda b, j, be, vr: (
                jnp.minimum(b, be[_NB] - 1), 0)),
            pl.BlockSpec((1, _IBLK, _H), lambda b, j, be, vr: (
                be[b], jnp.where(b < be[_NB], j, _NI - 1), 0)),
            pl.BlockSpec((1, _IBLK, _H), lambda b, j, be, vr: (
                be[b], jnp.where(b < be[_NB], j, _NI - 1), 0)),
            pl.BlockSpec((1, _H, _IBLK), lambda b, j, be, vr: (
                be[b], 0, jnp.where(b < be[_NB], j, _NI - 1))),
        ],
        out_specs=pl.BlockSpec((_BLK, _H), lambda b, j, be, vr: (
            jnp.minimum(b, be[_NB] - 1), 0)),
    )
    return pl.pallas_call(
        _ffn_body,
        grid_spec=grid_spec,
        out_shape=jax.ShapeDtypeStruct((_NROWS, _H), jnp.float32),
        compiler_params=pltpu.CompilerParams(
            dimension_semantics=("arbitrary", "arbitrary")),
    )(be, vr, xs, gate_w, up_w, down_w)


def _combine_body(outs_hbm, pos1_hbm, pos2_hbm, w1b_hbm, w2b_hbm, fin_hbm,
                  rows1_v, rows2_v, acc_v, idx1_v, idx2_v, w1_v, w2_v, sem):
    wid = lax.axis_index("s") * _NC + lax.axis_index("c")
    for c in range(_TPW // _CH):
        base = wid * _TPW + c * _CH
        pltpu.sync_copy(pos1_hbm.at[pl.ds(base, _CH)], idx1_v)
        pltpu.sync_copy(pos2_hbm.at[pl.ds(base, _CH)], idx2_v)
        pltpu.sync_copy(w1b_hbm.at[pl.ds(base, _CH)], w1_v)
        pltpu.sync_copy(w2b_hbm.at[pl.ds(base, _CH)], w2_v)
        cp1 = pltpu.async_copy(outs_hbm.at[idx1_v], rows1_v, sem)
        cp1.wait()
        cp2 = pltpu.async_copy(outs_hbm.at[idx2_v], rows2_v, sem)
        cp2.wait()

        def body(i, carry):
            wv1 = w1_v[i, :]
            wv2 = w2_v[i, :]
            for l in range(_H // 16):
                sl = pl.ds(l * 16, 16)
                acc_v[i, sl] = wv1 * rows1_v[i, sl] + wv2 * rows2_v[i, sl]
            return carry

        lax.fori_loop(0, _CH, body, 0)
        pltpu.sync_copy(acc_v, fin_hbm.at[pl.ds(base, _CH)])


def _combine(outs, pos1, pos2, w1b, w2b):
    mesh = plsc.VectorSubcoreMesh(core_axis_name="c", subcore_axis_name="s",
                                  num_cores=_NC, num_subcores=_NS)
    return pl.kernel(
        _combine_body,
        out_type=jax.ShapeDtypeStruct((_S, _H), jnp.float32),
        mesh=mesh,
        scratch_types=[
            pltpu.VMEM((_CH, _H), jnp.float32),
            pltpu.VMEM((_CH, _H), jnp.float32),
            pltpu.VMEM((_CH, _H), jnp.float32),
            pltpu.VMEM((_CH,), jnp.int32),
            pltpu.VMEM((_CH,), jnp.int32),
            pltpu.VMEM((_CH, 16), jnp.float32),
            pltpu.VMEM((_CH, 16), jnp.float32),
            pltpu.SemaphoreType.DMA,
        ],
    )(outs, pos1, pos2, w1b, w2b)


@jax.jit
def kernel(x, router_w, router_b, gate_w, up_w, down_w):
    b, s, h = x.shape
    xf = x.reshape(s, h)
    pos1, pos2, w1b, w2b, be, vr = _router(xf, router_w, router_b)
    pos1 = pos1.reshape(_S)
    pos2 = pos2.reshape(_S)
    be = be.reshape(128)[:_NB + 1]
    vr = vr.reshape(128)[:_NB]
    xs = _dispatch(xf, pos1, pos2)
    outs = _ffn(be, vr, xs, gate_w, up_w, down_w)
    fin = _combine(outs, pos1, pos2, w1b, w2b)
    return fin.reshape(b, s, h)


# speedup vs baseline: 1.1642x; 1.0206x over previous
import functools

import jax
import jax.numpy as jnp
from jax import lax
from jax.experimental import pallas as pl
from jax.experimental.pallas import tpu as pltpu
from jax.experimental.pallas import tpu_sc as plsc

_S = 2048
_H = 1024
_I = 4096
_E = 8
_K = 2

_BLK = 512
_BLK_SHIFT = 9
_NB = 16
_NROWS = _NB * _BLK
_IBLK = 1024
_NI = _I // _IBLK

_NC, _NS = 2, 16
_NW = _NC * _NS
_TPW = _S // _NW
_CH = 32


def _router_body(xf_ref, rw_ref, rb_ref, pos1_ref, pos2_ref, w1b_ref,
                 w2b_ref, be_ref):
    xf = xf_ref[...]
    rw = rw_ref[...]
    rb = rb_ref[...]
    logits = lax.dot_general(xf, rw, (((1,), (1,)), ((), ())),
                             preferred_element_type=jnp.float32) + rb
    m = jnp.max(logits, axis=1, keepdims=True)
    ex = jnp.exp(logits - m)
    sm = ex / jnp.sum(ex, axis=1, keepdims=True)

    lane_e = lax.broadcasted_iota(jnp.int32, (_S, _E), 1)
    v1 = jnp.max(sm, axis=1, keepdims=True)
    i1 = jnp.min(jnp.where(sm == v1, lane_e, _E), axis=1, keepdims=True)
    sm2 = jnp.where(lane_e == i1, -1.0, sm)
    v2 = jnp.max(sm2, axis=1, keepdims=True)
    i2 = jnp.min(jnp.where(sm2 == v2, lane_e, _E), axis=1, keepdims=True)
    den = v1 + v2
    w1b_ref[...] = jnp.broadcast_to(v1 / den, (_S, 16))
    w2b_ref[...] = jnp.broadcast_to(v2 / den, (_S, 16))

    oh1 = (lane_e == i1)
    oh2 = (lane_e == i2)
    oh = (oh1 | oh2).astype(jnp.float32)

    rows = lax.broadcasted_iota(jnp.int32, (128, 128), 0)
    cols = lax.broadcasted_iota(jnp.int32, (128, 128), 1)
    tstrict = (rows > cols).astype(jnp.float32)
    parts = []
    carry = jnp.zeros((1, _E), jnp.float32)
    for b in range(_S // 128):
        blk = oh[b * 128:(b + 1) * 128, :]
        exc = lax.dot_general(tstrict, blk, (((1,), (0,)), ((), ())),
                              preferred_element_type=jnp.float32)
        parts.append(exc + carry)
        carry = carry + jnp.sum(blk, axis=0, keepdims=True)
    cum = jnp.concatenate(parts, axis=0)
    counts_i = carry.astype(jnp.int32)

    nb = (counts_i + (_BLK - 1)) >> _BLK_SHIFT
    nb_f = nb.astype(jnp.float32)
    e_r = lax.broadcasted_iota(jnp.int32, (_E, _E), 0)
    e_c = lax.broadcasted_iota(jnp.int32, (_E, _E), 1)
    u_lt = (e_r < e_c).astype(jnp.float32)
    u_le = (e_r <= e_c).astype(jnp.float32)
    cum_excl = lax.dot_general(nb_f, u_lt, (((1,), (0,)), ((), ())),
                               preferred_element_type=jnp.float32)
    cum_incl = lax.dot_general(nb_f, u_le, (((1,), (0,)), ((), ())),
                               preferred_element_type=jnp.float32)
    off_row = cum_excl * float(_BLK)

    dest = off_row + cum
    pos1 = jnp.sum(jnp.where(oh1, dest, 0.0), axis=1, keepdims=True)
    pos2 = jnp.sum(jnp.where(oh2, dest, 0.0), axis=1, keepdims=True)
    pos1_ref[...] = pos1.astype(jnp.int32)
    pos2_ref[...] = pos2.astype(jnp.int32)

    b_iota = lax.broadcasted_iota(jnp.int32, (1, 128), 1)
    cum_incl_i = cum_incl.astype(jnp.int32)
    be_acc = jnp.zeros((1, 128), jnp.int32)
    for e in range(_E):
        be_acc = be_acc + jnp.where(b_iota >= cum_incl_i[:, e:e + 1], 1, 0)
    lane8 = lax.broadcasted_iota(jnp.int32, (1, _E), 1)
    last_e = jnp.max(jnp.where(counts_i > 0, lane8, 0), axis=1, keepdims=True)
    used = cum_incl_i[:, _E - 1:_E]
    be = jnp.minimum(be_acc, last_e)
    be_ref[...] = jnp.where(b_iota == _NB, used, be)


def _router(xf, router_w, router_b):
    return pl.pallas_call(
        _router_body,
        out_shape=[
            jax.ShapeDtypeStruct((_S, 1), jnp.int32),
            jax.ShapeDtypeStruct((_S, 1), jnp.int32),
            jax.ShapeDtypeStruct((_S, 16), jnp.float32),
            jax.ShapeDtypeStruct((_S, 16), jnp.float32),
            jax.ShapeDtypeStruct((1, 128), jnp.int32),
        ],
    )(xf, router_w, router_b.reshape(1, _E))


def _dispatch_body(xf_hbm, pos1_hbm, pos2_hbm, xs_hbm, rows_v, idx1_v,
                   idx2_v, sem):
    wid = lax.axis_index("s") * _NC + lax.axis_index("c")
    base = wid * _TPW
    pltpu.sync_copy(xf_hbm.at[pl.ds(base, _TPW)], rows_v)
    pltpu.sync_copy(pos1_hbm.at[pl.ds(base, _TPW)], idx1_v)
    pltpu.sync_copy(pos2_hbm.at[pl.ds(base, _TPW)], idx2_v)
    pltpu.async_copy(rows_v, xs_hbm.at[idx1_v], sem).wait()
    pltpu.async_copy(rows_v, xs_hbm.at[idx2_v], sem).wait()


def _dispatch(xf, pos1, pos2):
    mesh = plsc.VectorSubcoreMesh(core_axis_name="c", subcore_axis_name="s",
                                  num_cores=_NC, num_subcores=_NS)
    return pl.kernel(
        _dispatch_body,
        out_type=jax.ShapeDtypeStruct((_NROWS, _H), jnp.float32),
        mesh=mesh,
        scratch_types=[
            pltpu.VMEM((_TPW, _H), jnp.float32),
            pltpu.VMEM((_TPW,), jnp.int32),
            pltpu.VMEM((_TPW,), jnp.int32),
            pltpu.SemaphoreType.DMA,
        ],
    )(xf, pos1, pos2)


def _ffn_body(be_ref, xs_ref, gw_ref, uw_ref, dw_ref, out_ref):
    b = pl.program_id(0)
    j = pl.program_id(1)
    used = be_ref[_NB]

    @pl.when(b < used)
    def _():
        xb = xs_ref[...].astype(jnp.bfloat16)
        gw = gw_ref[0].astype(jnp.bfloat16)
        uw = uw_ref[0].astype(jnp.bfloat16)
        dw = dw_ref[0].astype(jnp.bfloat16)
        g = lax.dot_general(xb, gw, (((1,), (1,)), ((), ())),
                            preferred_element_type=jnp.float32)
        u = lax.dot_general(xb, uw, (((1,), (1,)), ((), ())),
                            preferred_element_type=jnp.float32)
        h = (g * lax.logistic(g) * u).astype(jnp.bfloat16)
        p = lax.dot_general(h, dw, (((1,), (1,)), ((), ())),
                            preferred_element_type=jnp.float32)

        @pl.when(j == 0)
        def _():
            out_ref[...] = p

        @pl.when(j > 0)
        def _():
            out_ref[...] = out_ref[...] + p


def _ffn(be, xs, gate_w, up_w, down_w):
    grid_spec = pltpu.PrefetchScalarGridSpec(
        num_scalar_prefetch=1,
        grid=(_NB, _NI),
        in_specs=[
            pl.BlockSpec((_BLK, _H), lambda b, j, be: (b, 0)),
            pl.BlockSpec((1, _IBLK, _H), lambda b, j, be: (
                be[b], jnp.where(b < be[_NB], j, _NI - 1), 0)),
            pl.BlockSpec((1, _IBLK, _H), lambda b, j, be: (
                be[b], jnp.where(b < be[_NB], j, _NI - 1), 0)),
            pl.BlockSpec((1, _H, _IBLK), lambda b, j, be: (
                be[b], 0, jnp.where(b < be[_NB], j, _NI - 1))),
        ],
        out_specs=pl.BlockSpec((_BLK, _H), lambda b, j, be: (b, 0)),
    )
    return pl.pallas_call(
        _ffn_body,
        grid_spec=grid_spec,
        out_shape=jax.ShapeDtypeStruct((_NROWS, _H), jnp.float32),
        compiler_params=pltpu.CompilerParams(
            dimension_semantics=("arbitrary", "arbitrary")),
    )(be, xs, gate_w, up_w, down_w)


def _combine_body(outs_hbm, pos1_hbm, pos2_hbm, w1b_hbm, w2b_hbm, fin_hbm,
                  rows1_v, rows2_v, acc_v, idx1_v, idx2_v, w1_v, w2_v, sem):
    wid = lax.axis_index("s") * _NC + lax.axis_index("c")
    for c in range(_TPW // _CH):
        base = wid * _TPW + c * _CH
        pltpu.sync_copy(pos1_hbm.at[pl.ds(base, _CH)], idx1_v)
        pltpu.sync_copy(pos2_hbm.at[pl.ds(base, _CH)], idx2_v)
        pltpu.sync_copy(w1b_hbm.at[pl.ds(base, _CH)], w1_v)
        pltpu.sync_copy(w2b_hbm.at[pl.ds(base, _CH)], w2_v)
        cp1 = pltpu.async_copy(outs_hbm.at[idx1_v], rows1_v, sem)
        cp2 = pltpu.async_copy(outs_hbm.at[idx2_v], rows2_v, sem)
        cp1.wait()
        cp2.wait()

        def body(i, carry):
            wv1 = w1_v[i, :]
            wv2 = w2_v[i, :]
            for l in range(_H // 16):
                sl = pl.ds(l * 16, 16)
                acc_v[i, sl] = wv1 * rows1_v[i, sl] + wv2 * rows2_v[i, sl]
            return carry

        lax.fori_loop(0, _CH, body, 0)
        pltpu.sync_copy(acc_v, fin_hbm.at[pl.ds(base, _CH)])


def _combine(outs, pos1, pos2, w1b, w2b):
    mesh = plsc.VectorSubcoreMesh(core_axis_name="c", subcore_axis_name="s",
                                  num_cores=_NC, num_subcores=_NS)
    return pl.kernel(
        _combine_body,
        out_type=jax.ShapeDtypeStruct((_S, _H), jnp.float32),
        mesh=mesh,
        scratch_types=[
            pltpu.VMEM((_CH, _H), jnp.float32),
            pltpu.VMEM((_CH, _H), jnp.float32),
            pltpu.VMEM((_CH, _H), jnp.float32),
            pltpu.VMEM((_CH,), jnp.int32),
            pltpu.VMEM((_CH,), jnp.int32),
            pltpu.VMEM((_CH, 16), jnp.float32),
            pltpu.VMEM((_CH, 16), jnp.float32),
            pltpu.SemaphoreType.DMA,
        ],
    )(outs, pos1, pos2, w1b, w2b)


@jax.jit
def kernel(x, router_w, router_b, gate_w, up_w, down_w):
    b, s, h = x.shape
    xf = x.reshape(s, h)
    pos1, pos2, w1b, w2b, be = _router(xf, router_w, router_b)
    pos1 = pos1.reshape(_S)
    pos2 = pos2.reshape(_S)
    be = be.reshape(128)[:_NB + 1]
    xs = _dispatch(xf, pos1, pos2)
    outs = _ffn(be, xs, gate_w, up_w, down_w)
    fin = _combine(outs, pos1, pos2, w1b, w2b)
    return fin.reshape(b, s, h)
